# baseline (speedup 1.0000x reference)
import functools

import jax
import jax.numpy as jnp
from jax import lax
from jax.experimental import pallas as pl
from jax.experimental.pallas import tpu as pltpu
from jax.experimental.pallas import tpu_sc as plsc

NUM_SEG = 100000
N_CHILD = 100000
E_TOT = 6400000

NC = 2
NS = 16
NW = NC * NS
PW = E_TOT // NW
WIN = 10000
NWIN = PW // WIN
HSPL = 4000

NUM_PAD = 100096
ZCH = NUM_PAD // NS
CH_PAD = 100096


def _sc_partials(child_val, indices, segment_ids):
    mesh = plsc.VectorSubcoreMesh(core_axis_name="c", subcore_axis_name="s")

    @functools.partial(
        pl.kernel,
        mesh=mesh,
        out_type=jax.ShapeDtypeStruct((NC, NUM_PAD), jnp.float32),
        scratch_types=[
            pltpu.VMEM((WIN,), jnp.int32),
            pltpu.VMEM((WIN,), jnp.int32),
            pltpu.VMEM((WIN,), jnp.int32),
            pltpu.VMEM((WIN,), jnp.int32),
            pltpu.VMEM((WIN,), jnp.float32),
            pltpu.VMEM((WIN,), jnp.float32),
            pltpu.VMEM_SHARED((NUM_PAD,), jnp.float32),
            pltpu.VMEM_SHARED((CH_PAD,), jnp.float32),
            pltpu.SemaphoreType.DMA,
            pltpu.SemaphoreType.DMA,
            pltpu.SemaphoreType.DMA,
            pltpu.SemaphoreType.DMA,
            pltpu.SemaphoreType.DMA,
            pltpu.SemaphoreType.DMA,
        ],
    )
    def k(child_hbm, idx_hbm, seg_hbm, out_hbm, idx_v0, idx_v1, seg_v0,
          seg_v1, val_v0, val_v1, acc_sh, child_sh, sem0, sem1, semg,
          semh, semc0, semc1):
        z_v = val_v0.at[pl.ds(0, ZCH)]
        c = lax.axis_index("c")
        s = lax.axis_index("s")
        wid = s * NC + c

        def zbody(i, carry):
            z_v[pl.ds(i * 16, 16)] = jnp.zeros((16,), jnp.float32)
            return carry

        lax.fori_loop(0, ZCH // 16, zbody, 0)
        pltpu.sync_copy(z_v, acc_sh.at[pl.ds(s * ZCH, ZCH)])
        pltpu.sync_copy(child_hbm.at[pl.ds(s * ZCH, ZCH)], z_v)
        pltpu.sync_copy(z_v, child_sh.at[pl.ds(s * ZCH, ZCH)])
        plsc.subcore_barrier()

        base = wid * PW
        slots = ((idx_v0, seg_v0, val_v0, sem0, semc0),
                 (idx_v1, seg_v1, val_v1, sem1, semc1))

        def start_loads(w, slot):
            idx_v, seg_v, _, sem, _ = slots[slot]
            off = base + w * WIN
            pltpu.make_async_copy(idx_hbm.at[pl.ds(off, WIN)], idx_v,
                                  sem).start()
            pltpu.make_async_copy(seg_hbm.at[pl.ds(off, WIN)], seg_v,
                                  sem).start()

        def wait_loads(w, slot):
            idx_v, seg_v, _, sem, _ = slots[slot]
            off = base + w * WIN
            pltpu.make_async_copy(idx_hbm.at[pl.ds(off, WIN)], idx_v,
                                  sem).wait()
            pltpu.make_async_copy(seg_hbm.at[pl.ds(off, WIN)], seg_v,
                                  sem).wait()

        def wait_scatter(slot):
            _, seg_v, val_v, _, semc = slots[slot]
            pltpu.make_async_copy(val_v, acc_sh.at[seg_v], semc).wait()

        def gather_win(slot):
            idx_v, _, val_v, _, _ = slots[slot]
            c1 = pltpu.async_copy(
                child_sh.at[idx_v.at[pl.ds(0, HSPL)]],
                val_v.at[pl.ds(0, HSPL)], semh)
            c2 = pltpu.async_copy(
                child_sh.at[idx_v.at[pl.ds(HSPL, WIN - HSPL)]],
                val_v.at[pl.ds(HSPL, WIN - HSPL)], semg)
            c1.wait()
            c2.wait()

        def start_scatter(slot):
            _, seg_v, val_v, _, semc = slots[slot]
            pltpu.async_copy(val_v, acc_sh.at[seg_v], semc, add=True)

        start_loads(0, 0)
        wait_loads(0, 0)
        gather_win(0)
        start_scatter(0)
        start_loads(1, 1)
        wait_loads(1, 1)
        gather_win(1)
        wait_scatter(0)
        start_scatter(1)
        start_loads(2, 0)

        def pipe(j, carry):
            w0 = 2 * j + 2
            wait_loads(w0, 0)
            gather_win(0)
            wait_scatter(1)
            start_scatter(0)
            start_loads(w0 + 1, 1)
            wait_loads(w0 + 1, 1)
            gather_win(1)
            wait_scatter(0)
            start_scatter(1)

            @pl.when(w0 + 2 < NWIN)
            def _():
                start_loads(w0 + 2, 0)

            return carry

        lax.fori_loop(0, NWIN // 2 - 1, pipe, 0)
        wait_scatter(1)
        plsc.subcore_barrier()

        @pl.when(s == 0)
        def _():
            pltpu.sync_copy(acc_sh, out_hbm.at[c])

    return k(child_val, indices, segment_ids)


def _combine(partials):
    def body(p_ref, o_ref):
        o_ref[...] = p_ref[0:1, :] + p_ref[1:2, :]

    return pl.pallas_call(
        body,
        out_shape=jax.ShapeDtypeStruct((1, NUM_PAD), jnp.float32),
    )(partials)


@jax.jit
def kernel(child_val, indices, segment_ids):
    idx = indices.astype(jnp.int32)
    seg = segment_ids.astype(jnp.int32)
    child_pad = jnp.zeros((CH_PAD,), jnp.float32).at[:N_CHILD].set(child_val)
    partials = _sc_partials(child_pad, idx, seg)
    return _combine(partials)[:, :NUM_SEG]


# speedup vs baseline: 1.0036x; 1.0036x over previous
import functools

import jax
import jax.numpy as jnp
from jax import lax
from jax.experimental import pallas as pl
from jax.experimental.pallas import tpu as pltpu
from jax.experimental.pallas import tpu_sc as plsc

NUM_SEG = 100000
N_CHILD = 100000
E_TOT = 6400000

NC = 2
NS = 16
NW = NC * NS
PW = E_TOT // NW
WIN = 10000
NWIN = PW // WIN
HSPL = 5600

NUM_PAD = 100096
ZCH = NUM_PAD // NS
CH_PAD = 100096


def _sc_partials(child_val, indices, segment_ids):
    mesh = plsc.VectorSubcoreMesh(core_axis_name="c", subcore_axis_name="s")

    @functools.partial(
        pl.kernel,
        mesh=mesh,
        out_type=jax.ShapeDtypeStruct((NC, NUM_PAD), jnp.float32),
        scratch_types=[
            pltpu.VMEM((WIN,), jnp.int32),
            pltpu.VMEM((WIN,), jnp.int32),
            pltpu.VMEM((WIN,), jnp.int32),
            pltpu.VMEM((WIN,), jnp.int32),
            pltpu.VMEM((WIN,), jnp.float32),
            pltpu.VMEM((WIN,), jnp.float32),
            pltpu.VMEM_SHARED((NUM_PAD,), jnp.float32),
            pltpu.VMEM_SHARED((CH_PAD,), jnp.float32),
            pltpu.SemaphoreType.DMA,
            pltpu.SemaphoreType.DMA,
            pltpu.SemaphoreType.DMA,
            pltpu.SemaphoreType.DMA,
            pltpu.SemaphoreType.DMA,
            pltpu.SemaphoreType.DMA,
        ],
    )
    def k(child_hbm, idx_hbm, seg_hbm, out_hbm, idx_v0, idx_v1, seg_v0,
          seg_v1, val_v0, val_v1, acc_sh, child_sh, sem0, sem1, semg,
          semh, semc0, semc1):
        z_v = val_v0.at[pl.ds(0, ZCH)]
        c = lax.axis_index("c")
        s = lax.axis_index("s")
        wid = s * NC + c

        def zbody(i, carry):
            z_v[pl.ds(i * 16, 16)] = jnp.zeros((16,), jnp.float32)
            return carry

        lax.fori_loop(0, ZCH // 16, zbody, 0)
        pltpu.sync_copy(z_v, acc_sh.at[pl.ds(s * ZCH, ZCH)])
        pltpu.sync_copy(child_hbm.at[pl.ds(s * ZCH, ZCH)], z_v)
        pltpu.sync_copy(z_v, child_sh.at[pl.ds(s * ZCH, ZCH)])
        plsc.subcore_barrier()

        base = wid * PW
        slots = ((idx_v0, seg_v0, val_v0, sem0, semc0),
                 (idx_v1, seg_v1, val_v1, sem1, semc1))

        def start_loads(w, slot):
            idx_v, seg_v, _, sem, _ = slots[slot]
            off = base + w * WIN
            pltpu.make_async_copy(idx_hbm.at[pl.ds(off, WIN)], idx_v,
                                  sem).start()
            pltpu.make_async_copy(seg_hbm.at[pl.ds(off, WIN)], seg_v,
                                  sem).start()

        def wait_loads(w, slot):
            idx_v, seg_v, _, sem, _ = slots[slot]
            off = base + w * WIN
            pltpu.make_async_copy(idx_hbm.at[pl.ds(off, WIN)], idx_v,
                                  sem).wait()
            pltpu.make_async_copy(seg_hbm.at[pl.ds(off, WIN)], seg_v,
                                  sem).wait()

        def wait_scatter(slot):
            _, seg_v, val_v, _, semc = slots[slot]
            pltpu.make_async_copy(val_v, acc_sh.at[seg_v], semc).wait()

        def gather_win(slot):
            idx_v, _, val_v, _, _ = slots[slot]
            c1 = pltpu.async_copy(
                child_hbm.at[idx_v.at[pl.ds(0, HSPL)]],
                val_v.at[pl.ds(0, HSPL)], semh)
            c2 = pltpu.async_copy(
                child_sh.at[idx_v.at[pl.ds(HSPL, WIN - HSPL)]],
                val_v.at[pl.ds(HSPL, WIN - HSPL)], semg)
            c1.wait()
            c2.wait()

        def start_scatter(slot):
            _, seg_v, val_v, _, semc = slots[slot]
            pltpu.async_copy(val_v, acc_sh.at[seg_v], semc, add=True)

        start_loads(0, 0)
        wait_loads(0, 0)
        gather_win(0)
        start_scatter(0)
        start_loads(1, 1)
        wait_loads(1, 1)
        gather_win(1)
        wait_scatter(0)
        start_scatter(1)
        start_loads(2, 0)

        def pipe(j, carry):
            w0 = 2 * j + 2
            wait_loads(w0, 0)
            gather_win(0)
            wait_scatter(1)
            start_scatter(0)
            start_loads(w0 + 1, 1)
            wait_loads(w0 + 1, 1)
            gather_win(1)
            wait_scatter(0)
            start_scatter(1)

            @pl.when(w0 + 2 < NWIN)
            def _():
                start_loads(w0 + 2, 0)

            return carry

        lax.fori_loop(0, NWIN // 2 - 1, pipe, 0)
        wait_scatter(1)
        plsc.subcore_barrier()

        @pl.when(s == 0)
        def _():
            pltpu.sync_copy(acc_sh, out_hbm.at[c])

    return k(child_val, indices, segment_ids)


def _combine(partials):
    def body(p_ref, o_ref):
        o_ref[...] = p_ref[0:1, :] + p_ref[1:2, :]

    return pl.pallas_call(
        body,
        out_shape=jax.ShapeDtypeStruct((1, NUM_PAD), jnp.float32),
    )(partials)


@jax.jit
def kernel(child_val, indices, segment_ids):
    idx = indices.astype(jnp.int32)
    seg = segment_ids.astype(jnp.int32)
    child_pad = jnp.zeros((CH_PAD,), jnp.float32).at[:N_CHILD].set(child_val)
    partials = _sc_partials(child_pad, idx, seg)
    return _combine(partials)[:, :NUM_SEG]


# speedup vs baseline: 1.1050x; 1.1010x over previous
import functools

import jax
import jax.numpy as jnp
from jax import lax
from jax.experimental import pallas as pl
from jax.experimental.pallas import tpu as pltpu
from jax.experimental.pallas import tpu_sc as plsc

NUM_SEG = 100000
N_CHILD = 100000
E_TOT = 6400000

NC = 2
NS = 16
NW = NC * NS
PW = E_TOT // NW
WIN = 10000
NWIN = PW // WIN
HSPL = 2400

NUM_PAD = 100096
ZCH = NUM_PAD // NS
CH_PAD = 100096


def _sc_partials(child_val, indices, segment_ids):
    mesh = plsc.VectorSubcoreMesh(core_axis_name="c", subcore_axis_name="s")

    @functools.partial(
        pl.kernel,
        mesh=mesh,
        out_type=jax.ShapeDtypeStruct((NC, NUM_PAD), jnp.float32),
        scratch_types=[
            pltpu.VMEM((WIN,), jnp.int32),
            pltpu.VMEM((WIN,), jnp.int32),
            pltpu.VMEM((WIN,), jnp.int32),
            pltpu.VMEM((WIN,), jnp.int32),
            pltpu.VMEM((WIN,), jnp.float32),
            pltpu.VMEM((WIN,), jnp.float32),
            pltpu.VMEM_SHARED((NUM_PAD,), jnp.float32),
            pltpu.VMEM_SHARED((CH_PAD,), jnp.float32),
            pltpu.SemaphoreType.DMA,
            pltpu.SemaphoreType.DMA,
            pltpu.SemaphoreType.DMA,
            pltpu.SemaphoreType.DMA,
            pltpu.SemaphoreType.DMA,
            pltpu.SemaphoreType.DMA,
        ],
    )
    def k(child_hbm, idx_hbm, seg_hbm, out_hbm, idx_v0, idx_v1, seg_v0,
          seg_v1, val_v0, val_v1, acc_sh, child_sh, sem0, sem1, semg,
          semh, semc0, semc1):
        z_v = val_v0.at[pl.ds(0, ZCH)]
        c = lax.axis_index("c")
        s = lax.axis_index("s")
        wid = s * NC + c

        def zbody(i, carry):
            z_v[pl.ds(i * 16, 16)] = jnp.zeros((16,), jnp.float32)
            return carry

        lax.fori_loop(0, ZCH // 16, zbody, 0)
        pltpu.sync_copy(z_v, acc_sh.at[pl.ds(s * ZCH, ZCH)])
        pltpu.sync_copy(child_hbm.at[pl.ds(s * ZCH, ZCH)], z_v)
        pltpu.sync_copy(z_v, child_sh.at[pl.ds(s * ZCH, ZCH)])
        plsc.subcore_barrier()

        base = wid * PW
        slots = ((idx_v0, seg_v0, val_v0, sem0, semc0),
                 (idx_v1, seg_v1, val_v1, sem1, semc1))

        def start_loads(w, slot):
            idx_v, seg_v, _, sem, _ = slots[slot]
            off = base + w * WIN
            pltpu.make_async_copy(idx_hbm.at[pl.ds(off, WIN)], idx_v,
                                  sem).start()
            pltpu.make_async_copy(seg_hbm.at[pl.ds(off, WIN)], seg_v,
                                  sem).start()

        def wait_loads(w, slot):
            idx_v, seg_v, _, sem, _ = slots[slot]
            off = base + w * WIN
            pltpu.make_async_copy(idx_hbm.at[pl.ds(off, WIN)], idx_v,
                                  sem).wait()
            pltpu.make_async_copy(seg_hbm.at[pl.ds(off, WIN)], seg_v,
                                  sem).wait()

        def wait_scatter(slot):
            _, seg_v, val_v, _, semc = slots[slot]
            pltpu.make_async_copy(val_v, acc_sh.at[seg_v], semc).wait()

        def gather_win(slot):
            idx_v, _, val_v, _, _ = slots[slot]
            c1 = pltpu.async_copy(
                child_hbm.at[idx_v.at[pl.ds(0, HSPL)]],
                val_v.at[pl.ds(0, HSPL)], semh)
            c2 = pltpu.async_copy(
                child_sh.at[idx_v.at[pl.ds(HSPL, WIN - HSPL)]],
                val_v.at[pl.ds(HSPL, WIN - HSPL)], semg)
            c1.wait()
            c2.wait()

        def start_scatter(slot):
            _, seg_v, val_v, _, semc = slots[slot]
            pltpu.async_copy(val_v, acc_sh.at[seg_v], semc, add=True)

        start_loads(0, 0)
        wait_loads(0, 0)
        gather_win(0)
        start_scatter(0)
        start_loads(1, 1)
        wait_loads(1, 1)
        gather_win(1)
        wait_scatter(0)
        start_scatter(1)
        start_loads(2, 0)

        def pipe(j, carry):
            w0 = 2 * j + 2
            wait_loads(w0, 0)
            gather_win(0)
            wait_scatter(1)
            start_scatter(0)
            start_loads(w0 + 1, 1)
            wait_loads(w0 + 1, 1)
            gather_win(1)
            wait_scatter(0)
            start_scatter(1)

            @pl.when(w0 + 2 < NWIN)
            def _():
                start_loads(w0 + 2, 0)

            return carry

        lax.fori_loop(0, NWIN // 2 - 1, pipe, 0)
        wait_scatter(1)
        plsc.subcore_barrier()

        @pl.when(s == 0)
        def _():
            pltpu.sync_copy(acc_sh, out_hbm.at[c])

    return k(child_val, indices, segment_ids)


def _combine(partials):
    def body(p_ref, o_ref):
        o_ref[...] = p_ref[0:1, :] + p_ref[1:2, :]

    return pl.pallas_call(
        body,
        out_shape=jax.ShapeDtypeStruct((1, NUM_PAD), jnp.float32),
    )(partials)


@jax.jit
def kernel(child_val, indices, segment_ids):
    idx = indices.astype(jnp.int32)
    seg = segment_ids.astype(jnp.int32)
    child_pad = jnp.zeros((CH_PAD,), jnp.float32).at[:N_CHILD].set(child_val)
    partials = _sc_partials(child_pad, idx, seg)
    return _combine(partials)[:, :NUM_SEG]
